# Initial kernel scaffold; baseline (speedup 1.0000x reference)
import jax
import jax.numpy as jnp
from jax.experimental import pallas as pl


def kernel(x, indexes, labels):
    raise NotImplementedError("write your pallas kernel here")


# speedup vs baseline: 1.2169x; 1.2169x over previous
import functools

import jax
import jax.numpy as jnp
from jax import lax
from jax.experimental import pallas as pl
from jax.experimental.pallas import tpu as pltpu
from jax.experimental.pallas import tpu_sc as plsc


def _sc_gather(x_flat, indexes, labels, n):
    b = indexes.shape[0]
    nw = 32
    bpw = b // nw
    mesh = plsc.VectorSubcoreMesh(core_axis_name="c", subcore_axis_name="s")

    @functools.partial(
        pl.kernel,
        mesh=mesh,
        out_type=[
            jax.ShapeDtypeStruct((b,), jnp.int32),
            jax.ShapeDtypeStruct((b,), jnp.float32),
        ],
        scratch_types=[
            pltpu.VMEM((bpw,), jnp.int32),
            pltpu.VMEM((bpw,), jnp.int32),
            pltpu.VMEM((bpw,), jnp.int32),
            pltpu.VMEM((bpw,), jnp.float32),
            pltpu.SemaphoreType.DMA,
        ],
    )
    def k(x_hbm, idx_hbm, lab_hbm, y_hbm, xe_hbm, idx_v, fidx_v, y_v, xe_v, sem):
        wid = lax.axis_index("s") * 2 + lax.axis_index("c")
        base = wid * bpw
        pltpu.sync_copy(idx_hbm.at[pl.ds(base, bpw)], idx_v)
        for j in range(bpw // 16):
            v = idx_v[pl.ds(j * 16, 16)]
            rows = base + j * 16 + lax.iota(jnp.int32, 16)
            fidx_v[pl.ds(j * 16, 16)] = rows * n + v
        pltpu.async_copy(lab_hbm.at[idx_v], y_v, sem).wait()
        pltpu.async_copy(x_hbm.at[fidx_v], xe_v, sem).wait()
        pltpu.sync_copy(y_v, y_hbm.at[pl.ds(base, bpw)])
        pltpu.sync_copy(xe_v, xe_hbm.at[pl.ds(base, bpw)])

    return k(x_flat, indexes, labels)


def _nca_tc(x, labels_row, y_col, xe_col, block_w):
    b, n = x.shape
    nblk = pl.cdiv(n, block_w)
    tail = n - (nblk - 1) * block_w

    def body(x_ref, lab_ref, y_ref, xe_ref, out_ref, s_acc, p_acc):
        k = pl.program_id(0)

        @pl.when(k == 0)
        def _init():
            s_acc[...] = jnp.zeros_like(s_acc)
            p_acc[...] = jnp.zeros_like(p_acc)

        e = jnp.exp(x_ref[...])
        same = lab_ref[...] == y_ref[...]

        @pl.when(k < nblk - 1)
        def _full():
            s_acc[...] += jnp.sum(e, axis=1, keepdims=True)
            p_acc[...] += jnp.sum(jnp.where(same, e, 0.0), axis=1, keepdims=True)

        @pl.when(k == nblk - 1)
        def _last():
            valid = lax.broadcasted_iota(jnp.int32, (1, block_w), 1) < tail
            ev = jnp.where(valid, e, 0.0)
            s = s_acc[...] + jnp.sum(ev, axis=1, keepdims=True)
            psum = p_acc[...] + jnp.sum(jnp.where(same, ev, 0.0), axis=1,
                                        keepdims=True)
            eself = jnp.exp(xe_ref[...])
            p = psum - eself
            z = s - eself
            prob = p / z
            nz = prob != 0.0
            terms = jnp.where(nz, jnp.log(jnp.where(nz, prob, 1.0)), 0.0)
            out_ref[0, 0] = -jnp.sum(terms) / jnp.float32(b)

    out = pl.pallas_call(
        body,
        grid=(nblk,),
        in_specs=[
            pl.BlockSpec((b, block_w), lambda k: (0, k)),
            pl.BlockSpec((1, block_w), lambda k: (0, k)),
            pl.BlockSpec((b, 1), lambda k: (0, 0)),
            pl.BlockSpec((b, 1), lambda k: (0, 0)),
        ],
        out_specs=pl.BlockSpec((1, 1), lambda k: (0, 0),
                               memory_space=pltpu.SMEM),
        out_shape=jax.ShapeDtypeStruct((1, 1), jnp.float32),
        scratch_shapes=[
            pltpu.VMEM((b, 1), jnp.float32),
            pltpu.VMEM((b, 1), jnp.float32),
        ],
        compiler_params=pltpu.CompilerParams(
            dimension_semantics=("arbitrary",),
        ),
    )(x, labels_row, y_col, xe_col)
    return out[0, 0]


def kernel(x, indexes, labels):
    b, n = x.shape
    y, xe = _sc_gather(x.reshape(-1), indexes, labels, n)
    return _nca_tc(x, labels.reshape(1, -1), y.reshape(b, 1),
                   xe.reshape(b, 1), block_w=2048)


# speedup vs baseline: 2.4718x; 2.0311x over previous
import functools

import jax
import jax.numpy as jnp
from jax import lax
from jax.experimental import pallas as pl
from jax.experimental.pallas import tpu as pltpu
from jax.experimental.pallas import tpu_sc as plsc


def _sc_gather_y(indexes, labels):
    b = indexes.shape[0]
    nw = 32
    bpw = b // nw
    mesh = plsc.VectorSubcoreMesh(core_axis_name="c", subcore_axis_name="s")

    @functools.partial(
        pl.kernel,
        mesh=mesh,
        out_type=jax.ShapeDtypeStruct((b,), jnp.int32),
        scratch_types=[
            pltpu.VMEM((bpw,), jnp.int32),
            pltpu.VMEM((bpw,), jnp.int32),
            pltpu.SemaphoreType.DMA,
        ],
    )
    def k(idx_hbm, lab_hbm, y_hbm, idx_v, y_v, sem):
        wid = lax.axis_index("s") * 2 + lax.axis_index("c")
        base = wid * bpw
        pltpu.sync_copy(idx_hbm.at[pl.ds(base, bpw)], idx_v)
        pltpu.async_copy(lab_hbm.at[idx_v], y_v, sem).wait()
        pltpu.sync_copy(y_v, y_hbm.at[pl.ds(base, bpw)])

    return k(indexes, labels)


def _nca_tc(x, labels_row, y_col, idx_col, block_w):
    b, n = x.shape
    nblk = pl.cdiv(n, block_w)

    def body(x_ref, lab_ref, y_ref, idx_ref, out_ref, s_acc, p_acc):
        k = pl.program_id(0)

        @pl.when(k == 0)
        def _init():
            s_acc[...] = jnp.zeros_like(s_acc)
            p_acc[...] = jnp.zeros_like(p_acc)

        cols = lax.broadcasted_iota(jnp.int32, (1, block_w), 1) + k * block_w
        e = jnp.exp(x_ref[...])
        same = lab_ref[...] == y_ref[...]

        @pl.when(k < nblk - 1)
        def _full():
            e0 = jnp.where(cols == idx_ref[...], 0.0, e)
            s_acc[...] += jnp.sum(e0, axis=1, keepdims=True)
            p_acc[...] += jnp.sum(jnp.where(same, e0, 0.0), axis=1,
                                  keepdims=True)

        @pl.when(k == nblk - 1)
        def _last():
            kill = (cols == idx_ref[...]) | (cols >= n)
            e0 = jnp.where(kill, 0.0, e)
            z = s_acc[...] + jnp.sum(e0, axis=1, keepdims=True)
            p = p_acc[...] + jnp.sum(jnp.where(same, e0, 0.0), axis=1,
                                     keepdims=True)
            prob = p / z
            nz = prob != 0.0
            terms = jnp.where(nz, jnp.log(jnp.where(nz, prob, 1.0)), 0.0)
            out_ref[0, 0] = -jnp.sum(terms) / jnp.float32(b)

    out = pl.pallas_call(
        body,
        grid=(nblk,),
        in_specs=[
            pl.BlockSpec((b, block_w), lambda k: (0, k)),
            pl.BlockSpec((1, block_w), lambda k: (0, k)),
            pl.BlockSpec((b, 1), lambda k: (0, 0)),
            pl.BlockSpec((b, 1), lambda k: (0, 0)),
        ],
        out_specs=pl.BlockSpec((1, 1), lambda k: (0, 0),
                               memory_space=pltpu.SMEM),
        out_shape=jax.ShapeDtypeStruct((1, 1), jnp.float32),
        scratch_shapes=[
            pltpu.VMEM((b, 1), jnp.float32),
            pltpu.VMEM((b, 1), jnp.float32),
        ],
        compiler_params=pltpu.CompilerParams(
            dimension_semantics=("arbitrary",),
        ),
    )(x, labels_row, y_col, idx_col)
    return out[0, 0]


def kernel(x, indexes, labels):
    b, n = x.shape
    y = _sc_gather_y(indexes, labels)
    return _nca_tc(x, labels.reshape(1, -1), y.reshape(b, 1),
                   indexes.reshape(b, 1), block_w=2048)
